# Initial kernel scaffold; baseline (speedup 1.0000x reference)
import jax
import jax.numpy as jnp
from jax.experimental import pallas as pl


def kernel(sequence, node_enc_W, node_enc_b, edge_enc_W, edge_enc_b, conv1_W, conv1_root, conv1_b, conv2_W, conv2_root, conv2_b, fc1_W, fc1_b, fc15_W, fc15_b, fc2_W, fc2_b):
    raise NotImplementedError("write your pallas kernel here")


# speedup vs baseline: 5551.9441x; 5551.9441x over previous
import functools

import jax
import jax.numpy as jnp
from jax.experimental import pallas as pl
from jax.experimental.pallas import tpu as pltpu

T, N, R = 8, 256, 9
F_IN, F_NODE, HID = 15, 128, 128
LSTM1, LSTM2, NCLASS = 128, 64, 8
THRESH = 0.9


def _frame_kernel(seq_ref, new_ref, neb_ref, ew1_ref, ew2_ref, eb_ref,
                  c1w_ref, c1r_ref, c1b_ref, c2w_ref, c2r_ref, c2b_ref,
                  f1w_ref, f1b_ref, f15w_ref, f15b_ref, f2w_ref, f2b_ref,
                  out_ref, acc_ref):
    t = pl.program_id(0)
    nf = seq_ref[0]

    dot = functools.partial(jnp.dot, preferred_element_type=jnp.float32)

    x = jax.nn.relu(dot(nf, new_ref[...]) + neb_ref[...])

    u = dot(nf, ew1_ref[...])
    v = dot(nf, ew2_ref[...]) + eb_ref[...]
    vt = v.T

    ev = []
    for r in range(R):
        logits = u[:, r:r + 1] + vt[r:r + 1, :]
        ev.append(jax.nn.sigmoid(logits))

    mval = ev[0]
    midx = jnp.zeros((N, N), jnp.int32)
    for r in range(1, R):
        upd = ev[r] > mval
        midx = jnp.where(upd, r, midx)
        mval = jnp.where(upd, ev[r], mval)

    rows = jax.lax.broadcasted_iota(jnp.int32, (N, N), 0)
    cols = jax.lax.broadcasted_iota(jnp.int32, (N, N), 1)
    upper = cols > rows

    A = []
    inv = []
    for r in range(R):
        m = jnp.where(
            upper,
            (midx == r).astype(jnp.float32) + (ev[r] > THRESH).astype(jnp.float32),
            0.0,
        )
        a_r = m + m.T
        cnt = jnp.sum(a_r, axis=0)
        A.append(a_r)
        inv.append(1.0 / jnp.maximum(cnt, 1.0))

    def conv(xin, w_ref, root_ref, b_ref):
        agg = dot(xin, root_ref[...]) + b_ref[...]
        for r in range(R):
            h_r = dot(xin, w_ref[r])
            agg += dot(A[r], h_r) * inv[r][:, None]
        return jax.nn.relu(agg)

    h1 = conv(x, c1w_ref, c1r_ref, c1b_ref)
    h2 = conv(h1, c2w_ref, c2r_ref, c2b_ref)

    g = jnp.concatenate(
        [jnp.mean(h1, axis=0, keepdims=True), jnp.mean(h2, axis=0, keepdims=True)],
        axis=1,
    )
    gr = jax.nn.relu(dot(g, f1w_ref[...]) + f1b_ref[...])

    @pl.when(t == 0)
    def _():
        acc_ref[...] = gr

    @pl.when(t > 0)
    def _():
        acc_ref[...] += gr

    @pl.when(t == T - 1)
    def _():
        tm = acc_ref[...] * (1.0 / T)
        o = jax.nn.relu(dot(tm, f15w_ref[...]) + f15b_ref[...])
        out_ref[...] = dot(o, f2w_ref[...]) + f2b_ref[...]


def kernel(sequence, node_enc_W, node_enc_b, edge_enc_W, edge_enc_b,
           conv1_W, conv1_root, conv1_b, conv2_W, conv2_root, conv2_b,
           fc1_W, fc1_b, fc15_W, fc15_b, fc2_W, fc2_b):
    ew1 = edge_enc_W[:F_IN]
    ew2 = edge_enc_W[F_IN:]

    full = lambda shape: pl.BlockSpec(shape, lambda t: (0,) * len(shape))

    out = pl.pallas_call(
        _frame_kernel,
        grid=(T,),
        in_specs=[
            pl.BlockSpec((1, N, F_IN), lambda t: (t, 0, 0)),
            full((F_IN, F_NODE)),
            full((1, F_NODE)),
            full((F_IN, R)),
            full((F_IN, R)),
            full((1, R)),
            full((R, F_NODE, HID)),
            full((F_NODE, HID)),
            full((1, HID)),
            full((R, HID, HID)),
            full((HID, HID)),
            full((1, HID)),
            full((2 * HID, LSTM1)),
            full((1, LSTM1)),
            full((LSTM1, LSTM2)),
            full((1, LSTM2)),
            full((LSTM2, NCLASS)),
            full((1, NCLASS)),
        ],
        out_specs=pl.BlockSpec((1, NCLASS), lambda t: (0, 0)),
        out_shape=jax.ShapeDtypeStruct((1, NCLASS), jnp.float32),
        scratch_shapes=[pltpu.VMEM((1, LSTM1), jnp.float32)],
    )(
        sequence,
        node_enc_W, node_enc_b.reshape(1, F_NODE),
        ew1, ew2, edge_enc_b.reshape(1, R),
        conv1_W, conv1_root, conv1_b.reshape(1, HID),
        conv2_W, conv2_root, conv2_b.reshape(1, HID),
        fc1_W, fc1_b.reshape(1, LSTM1),
        fc15_W, fc15_b.reshape(1, LSTM2),
        fc2_W, fc2_b.reshape(1, NCLASS),
    )
    return out.reshape(NCLASS)
